# baseline (speedup 1.0000x reference)
import functools

import jax
import jax.numpy as jnp
from jax import lax
from jax.experimental import pallas as pl
from jax.experimental.pallas import tpu as pltpu


def _fused_body(labels_ref, features_ref, centroids_ref, theta_ref, theta_row_ref,
                loss_ref, radius_ref, f_acc, cnt_acc, *, batch_total):
    g = pl.program_id(0)
    nb = pl.num_programs(0)
    blk, num_labels = labels_ref.shape

    rad_row = jax.nn.softplus(theta_row_ref[...])

    @pl.when(g == 0)
    def _init():
        radius_ref[...] = jax.nn.softplus(theta_ref[...])
        f_acc[...] = jnp.zeros_like(f_acc)
        cnt_acc[...] = jnp.zeros_like(cnt_acc)

    lbl = labels_ref[...]
    col = lax.broadcasted_iota(jnp.int32, (blk, num_labels), 1)
    row_max = jnp.max(lbl, axis=1, keepdims=True)
    first = jnp.min(jnp.where(lbl == row_max, col, num_labels),
                    axis=1, keepdims=True)
    onehot = (col == first).astype(jnp.bfloat16)

    c = jnp.dot(onehot, centroids_ref[...],
                preferred_element_type=jnp.float32)
    diff = features_ref[...] - c
    d = jnp.sqrt(jnp.sum(diff * diff, axis=1, keepdims=True))

    cnt_acc[...] += jnp.sum(onehot, axis=0, keepdims=True)
    f_acc[...] += jnp.sum(jnp.abs(d - rad_row), axis=0, keepdims=True)

    @pl.when(g == nb - 1)
    def _final():
        total = jnp.sum(f_acc[...] * cnt_acc[...], axis=1, keepdims=True)
        loss_ref[...] = total[:, :1] / jnp.float32(batch_total * batch_total)


def kernel(features, centroids, labels, theta):
    batch, feat_dim = features.shape
    num_labels = centroids.shape[0]
    blk = 2048
    grid = (batch // blk,)

    theta_row = theta.reshape(1, num_labels)
    centroids_bf16 = centroids.astype(jnp.bfloat16)

    loss2d, radius = pl.pallas_call(
        functools.partial(_fused_body, batch_total=batch),
        grid=grid,
        in_specs=[
            pl.BlockSpec((blk, num_labels), lambda g: (g, 0)),
            pl.BlockSpec((blk, feat_dim), lambda g: (g, 0)),
            pl.BlockSpec((num_labels, feat_dim), lambda g: (0, 0)),
            pl.BlockSpec((num_labels, 1), lambda g: (0, 0)),
            pl.BlockSpec((1, num_labels), lambda g: (0, 0)),
        ],
        out_specs=(
            pl.BlockSpec((1, 1), lambda g: (0, 0)),
            pl.BlockSpec((num_labels, 1), lambda g: (0, 0)),
        ),
        out_shape=(
            jax.ShapeDtypeStruct((1, 1), jnp.float32),
            jax.ShapeDtypeStruct((num_labels, 1), jnp.float32),
        ),
        scratch_shapes=[
            pltpu.VMEM((1, num_labels), jnp.float32),
            pltpu.VMEM((1, num_labels), jnp.float32),
        ],
    )(labels, features, centroids_bf16, theta, theta_row)

    return loss2d[0, 0], radius


# speedup vs baseline: 1.0482x; 1.0482x over previous
import functools

import jax
import jax.numpy as jnp
from jax import lax
from jax.experimental import pallas as pl
from jax.experimental.pallas import tpu as pltpu


def _block_body(labels_ref, features_ref, centroids_ref, theta_row_ref,
                f_part_ref, cnt_part_ref):
    blk, num_labels = labels_ref.shape

    rad_row = jax.nn.softplus(theta_row_ref[...])

    lbl = labels_ref[...]
    col = lax.broadcasted_iota(jnp.int32, (blk, num_labels), 1)
    row_max = jnp.max(lbl, axis=1, keepdims=True)
    first = jnp.min(jnp.where(lbl == row_max, col, num_labels),
                    axis=1, keepdims=True)
    onehot = (col == first).astype(jnp.bfloat16)

    c = jnp.dot(onehot, centroids_ref[...],
                preferred_element_type=jnp.float32)
    diff = features_ref[...] - c
    d = jnp.sqrt(jnp.sum(diff * diff, axis=1, keepdims=True))

    cnt_part_ref[...] = jnp.sum(onehot.astype(jnp.float32), axis=0,
                                keepdims=True)[None]
    f_part_ref[...] = jnp.sum(jnp.abs(d - rad_row), axis=0, keepdims=True)[None]


def _reduce_body(f_part_ref, cnt_part_ref, theta_ref, loss_ref, radius_ref,
                 *, batch_total):
    radius_ref[...] = jax.nn.softplus(theta_ref[...])
    f_tot = jnp.sum(f_part_ref[...], axis=0)
    cnt_tot = jnp.sum(cnt_part_ref[...], axis=0)
    total = jnp.sum(f_tot * cnt_tot, axis=1, keepdims=True)
    loss_ref[...] = total / jnp.float32(batch_total * batch_total)


def kernel(features, centroids, labels, theta):
    batch, feat_dim = features.shape
    num_labels = centroids.shape[0]
    blk = 1024
    nb = batch // blk

    theta_row = theta.reshape(1, num_labels)
    centroids_bf16 = centroids.astype(jnp.bfloat16)

    f_part, cnt_part = pl.pallas_call(
        _block_body,
        grid=(nb,),
        in_specs=[
            pl.BlockSpec((blk, num_labels), lambda g: (g, 0)),
            pl.BlockSpec((blk, feat_dim), lambda g: (g, 0)),
            pl.BlockSpec((num_labels, feat_dim), lambda g: (0, 0)),
            pl.BlockSpec((1, num_labels), lambda g: (0, 0)),
        ],
        out_specs=(
            pl.BlockSpec((1, 1, num_labels), lambda g: (g, 0, 0)),
            pl.BlockSpec((1, 1, num_labels), lambda g: (g, 0, 0)),
        ),
        out_shape=(
            jax.ShapeDtypeStruct((nb, 1, num_labels), jnp.float32),
            jax.ShapeDtypeStruct((nb, 1, num_labels), jnp.float32),
        ),
        compiler_params=pltpu.CompilerParams(
            dimension_semantics=("parallel",),
        ),
    )(labels, features, centroids_bf16, theta_row)

    loss2d, radius = pl.pallas_call(
        functools.partial(_reduce_body, batch_total=batch),
        out_shape=(
            jax.ShapeDtypeStruct((1, 1), jnp.float32),
            jax.ShapeDtypeStruct((num_labels, 1), jnp.float32),
        ),
    )(f_part, cnt_part, theta)

    return loss2d[0, 0], radius
